# baseline (speedup 1.0000x reference)
import functools

import jax
import jax.numpy as jnp
from jax import lax
from jax.experimental import pallas as pl
from jax.experimental.pallas import tpu as pltpu
from jax.experimental.pallas import tpu_sc as plsc

N_NODES_C = 10000

NC = 2
NS = 16
NW = NC * NS
CHUNK = 128

ACC_ROWS = (N_NODES_C // 128 + 1) * 128
ROWS_PER_TILE = ACC_ROWS // NS


def _fill_zero(ref, n_rows, n_col16):
    zero = jnp.zeros((16,), jnp.float32)
    rpi = max(1, 8 // n_col16)

    def body(k, _):
        for ur in range(rpi):
            for uc in range(n_col16):
                ref[k * rpi + ur, pl.ds(uc * 16, 16)] = zero
        return 0

    lax.fori_loop(0, n_rows // rpi, body, 0)


def _fill_ones(ref, n_rows):
    one = jnp.ones((16,), jnp.float32)

    def body(k, _):
        for u in range(8):
            ref[k * 8 + u, :] = one
        return 0

    lax.fori_loop(0, n_rows // 8, body, 0)


def _make_deg_kernel(e_rows):
    rpt = e_rows // NW
    mesh = plsc.VectorSubcoreMesh(core_axis_name="c", subcore_axis_name="s")

    @functools.partial(
        pl.kernel,
        mesh=mesh,
        out_type=jax.ShapeDtypeStruct((NC, ACC_ROWS, 16), jnp.float32),
        scratch_types=[
            pltpu.VMEM((rpt, CHUNK), jnp.int32),
            pltpu.VMEM((CHUNK, 16), jnp.float32),
            pltpu.VMEM((ROWS_PER_TILE, 16), jnp.float32),
            pltpu.VMEM_SHARED((ACC_ROWS, 16), jnp.float32),
        ],
        compiler_params=pltpu.CompilerParams(use_tc_tiling_on_sc=False),
    )
    def deg_kernel(dst_hbm, out_hbm, dst_v, ones_v, stage_v, acc_sh):
        c = lax.axis_index("c")
        s = lax.axis_index("s")
        t = c * NS + s
        pltpu.sync_copy(dst_hbm.at[pl.ds(t * rpt, rpt)], dst_v)
        _fill_ones(ones_v, CHUNK)
        _fill_zero(stage_v, ROWS_PER_TILE, 1)
        pltpu.sync_copy(stage_v, acc_sh.at[pl.ds(s * ROWS_PER_TILE, ROWS_PER_TILE)])
        plsc.subcore_barrier()

        def body(j, _):
            pltpu.sync_copy(ones_v, acc_sh.at[dst_v.at[j]], add=True)
            return 0

        lax.fori_loop(0, rpt, body, 0)
        plsc.subcore_barrier()
        pltpu.sync_copy(acc_sh.at[pl.ds(s * ROWS_PER_TILE, ROWS_PER_TILE)], stage_v)
        pltpu.sync_copy(stage_v, out_hbm.at[c, pl.ds(s * ROWS_PER_TILE, ROWS_PER_TILE)])

    return deg_kernel


def _make_prop_kernel(e_rows, d):
    rpt = e_rows // NW
    ncol16 = d // 16
    mesh = plsc.VectorSubcoreMesh(core_axis_name="c", subcore_axis_name="s")

    @functools.partial(
        pl.kernel,
        mesh=mesh,
        out_type=jax.ShapeDtypeStruct((NC, ACC_ROWS, d), jnp.float32),
        scratch_types=[
            pltpu.VMEM((rpt + 2, CHUNK), jnp.int32),
            pltpu.VMEM((rpt, CHUNK), jnp.int32),
            pltpu.VMEM((CHUNK, d), jnp.float32),
            pltpu.VMEM((CHUNK, d), jnp.float32),
            pltpu.VMEM((ROWS_PER_TILE, d), jnp.float32),
            pltpu.VMEM_SHARED((ACC_ROWS, d), jnp.float32),
            pltpu.SemaphoreType.DMA,
            pltpu.SemaphoreType.DMA,
        ],
        compiler_params=pltpu.CompilerParams(use_tc_tiling_on_sc=False),
    )
    def prop_kernel(src_hbm, dst_hbm, g_hbm, out_hbm,
                    src_v, dst_v, rows0_v, rows1_v, stage_v, acc_sh,
                    sem0, sem1):
        c = lax.axis_index("c")
        s = lax.axis_index("s")
        t = c * NS + s
        pltpu.sync_copy(src_hbm.at[pl.ds(t * rpt, rpt)], src_v.at[pl.ds(0, rpt)])
        pltpu.sync_copy(dst_hbm.at[pl.ds(t * rpt, rpt)], dst_v)
        zero_i = jnp.zeros((16,), jnp.int32)
        for u in range(CHUNK // 16):
            src_v[rpt, pl.ds(u * 16, 16)] = zero_i
            src_v[rpt + 1, pl.ds(u * 16, 16)] = zero_i
        _fill_zero(stage_v, ROWS_PER_TILE, ncol16)
        pltpu.sync_copy(stage_v, acc_sh.at[pl.ds(s * ROWS_PER_TILE, ROWS_PER_TILE)])
        plsc.subcore_barrier()

        bufs = (rows0_v, rows1_v)
        sems = (sem0, sem1)
        pltpu.async_copy(g_hbm.at[src_v.at[0]], rows0_v, sem0)
        pltpu.async_copy(g_hbm.at[src_v.at[1]], rows1_v, sem1)

        def body(g, _):
            for b in range(2):
                j = g * 2 + b
                pltpu.make_async_copy(g_hbm.at[src_v.at[j]], bufs[b], sems[b]).wait()
                pltpu.sync_copy(bufs[b], acc_sh.at[dst_v.at[j]], add=True)
                pltpu.async_copy(g_hbm.at[src_v.at[j + 2]], bufs[b], sems[b])
            return 0

        lax.fori_loop(0, rpt // 2, body, 0)
        pltpu.make_async_copy(g_hbm.at[src_v.at[rpt]], rows0_v, sem0).wait()
        pltpu.make_async_copy(g_hbm.at[src_v.at[rpt + 1]], rows1_v, sem1).wait()
        plsc.subcore_barrier()
        pltpu.sync_copy(acc_sh.at[pl.ds(s * ROWS_PER_TILE, ROWS_PER_TILE)], stage_v)
        pltpu.sync_copy(stage_v, out_hbm.at[c, pl.ds(s * ROWS_PER_TILE, ROWS_PER_TILE)])

    return prop_kernel


_BN = 2000


def _l1_body(x_ref, w_ref, degp_ref, g1_ref, dis_ref):
    h = jnp.dot(x_ref[...], w_ref[...], preferred_element_type=jnp.float32)
    deg = degp_ref[0] + degp_ref[1] + 1.0
    dis = lax.rsqrt(deg)
    dis_ref[...] = dis
    g1_ref[...] = h * dis[:, 0:1]


def _mid_body(p_ref, g1_ref, dis_ref, b1_ref, g2_ref):
    d = dis_ref[:, 0:1]
    z = jnp.maximum(d * (p_ref[0] + p_ref[1] + g1_ref[...]) + b1_ref[...], 0.0)
    g2_ref[...] = d * z


def _out_body(p_ref, g2_ref, dis_ref, w_ref, b_ref, o_ref):
    agg = dis_ref[:, 0:1] * (p_ref[0] + p_ref[1] + g2_ref[...])
    o_ref[...] = (
        jnp.dot(agg, w_ref[...], preferred_element_type=jnp.float32) + b_ref[...]
    )


def kernel(x, edge_index, W1, b1, W2, b2):
    n, d_in = x.shape
    d_hid = W1.shape[1]
    d_out = W2.shape[1]
    e = edge_index.shape[1]

    ei = edge_index.astype(jnp.int32)
    e_pad = ((e + NW * CHUNK - 1) // (NW * CHUNK)) * (NW * CHUNK)
    pad = e_pad - e
    pad_iota = jnp.arange(pad, dtype=jnp.int32)
    src = jnp.concatenate([ei[0], pad_iota % n])
    dst = jnp.concatenate([ei[1], n + pad_iota % (ACC_ROWS - n)])
    e_rows = e_pad // CHUNK
    src2d = src.reshape(e_rows, CHUNK)
    dst2d = dst.reshape(e_rows, CHUNK)

    deg_call = _make_deg_kernel(e_rows)
    prop_call = _make_prop_kernel(e_rows, d_hid)

    grid = (n // _BN,)
    spec_rows = lambda w: pl.BlockSpec((_BN, w), lambda i: (i, 0))
    spec_pair = lambda w: pl.BlockSpec((2, _BN, w), lambda i: (0, i, 0))
    spec_full = lambda a, b: pl.BlockSpec((a, b), lambda i: (0, 0))

    degp = deg_call(dst2d)[:, :n, :]

    g1, dis = pl.pallas_call(
        _l1_body,
        grid=grid,
        in_specs=[spec_rows(d_in), spec_full(d_in, d_hid), spec_pair(16)],
        out_specs=[spec_rows(d_hid), spec_rows(16)],
        out_shape=[
            jax.ShapeDtypeStruct((n, d_hid), jnp.float32),
            jax.ShapeDtypeStruct((n, 16), jnp.float32),
        ],
    )(x, W1, degp)

    p1 = prop_call(src2d, dst2d, g1)[:, :n, :]

    g2 = pl.pallas_call(
        _mid_body,
        grid=grid,
        in_specs=[spec_pair(d_hid), spec_rows(d_hid), spec_rows(16),
                  spec_full(1, d_hid)],
        out_specs=spec_rows(d_hid),
        out_shape=jax.ShapeDtypeStruct((n, d_hid), jnp.float32),
    )(p1, g1, dis, b1.reshape(1, d_hid))

    p2 = prop_call(src2d, dst2d, g2)[:, :n, :]

    out = pl.pallas_call(
        _out_body,
        grid=grid,
        in_specs=[spec_pair(d_hid), spec_rows(d_hid), spec_rows(16),
                  spec_full(d_hid, d_out), spec_full(1, d_out)],
        out_specs=spec_rows(d_out),
        out_shape=jax.ShapeDtypeStruct((n, d_out), jnp.float32),
    )(p2, g2, dis, W2, b2.reshape(1, d_out))

    return out


# speedup vs baseline: 2.0414x; 2.0414x over previous
import functools

import jax
import jax.numpy as jnp
from jax import lax
from jax.experimental import pallas as pl
from jax.experimental.pallas import tpu as pltpu
from jax.experimental.pallas import tpu_sc as plsc

N_NODES_C = 10000

NC = 2
NS = 16
NW = NC * NS
CHUNK = 128

ACC_ROWS = (N_NODES_C // 128 + 1) * 128
ROWS_PER_TILE = ACC_ROWS // NS


def _fill_zero(ref, n_rows, n_col16):
    zero = jnp.zeros((16,), jnp.float32)
    rpi = max(1, 8 // n_col16)

    def body(k, _):
        for ur in range(rpi):
            for uc in range(n_col16):
                ref[k * rpi + ur, pl.ds(uc * 16, 16)] = zero
        return 0

    lax.fori_loop(0, n_rows // rpi, body, 0)


def _fill_ones(ref, n_rows):
    one = jnp.ones((16,), jnp.float32)

    def body(k, _):
        for u in range(8):
            ref[k * 8 + u, :] = one
        return 0

    lax.fori_loop(0, n_rows // 8, body, 0)


def _make_deg_kernel(e_rows):
    rpt = e_rows // NW
    mesh = plsc.VectorSubcoreMesh(core_axis_name="c", subcore_axis_name="s")

    @functools.partial(
        pl.kernel,
        mesh=mesh,
        out_type=jax.ShapeDtypeStruct((NC, ACC_ROWS, 16), jnp.float32),
        scratch_types=[
            pltpu.VMEM((rpt, CHUNK), jnp.int32),
            pltpu.VMEM((CHUNK, 16), jnp.float32),
            pltpu.VMEM((ROWS_PER_TILE, 16), jnp.float32),
            pltpu.VMEM_SHARED((ACC_ROWS, 16), jnp.float32),
        ],
        compiler_params=pltpu.CompilerParams(use_tc_tiling_on_sc=False),
    )
    def deg_kernel(dst_hbm, out_hbm, dst_v, ones_v, stage_v, acc_sh):
        c = lax.axis_index("c")
        s = lax.axis_index("s")
        t = c * NS + s
        pltpu.sync_copy(dst_hbm.at[pl.ds(t * rpt, rpt)], dst_v)
        _fill_ones(ones_v, CHUNK)
        _fill_zero(stage_v, ROWS_PER_TILE, 1)
        pltpu.sync_copy(stage_v, acc_sh.at[pl.ds(s * ROWS_PER_TILE, ROWS_PER_TILE)])
        plsc.subcore_barrier()

        def body(j, _):
            pltpu.sync_copy(ones_v, acc_sh.at[dst_v.at[j]], add=True)
            return 0

        lax.fori_loop(0, rpt, body, 0)
        plsc.subcore_barrier()
        pltpu.sync_copy(acc_sh.at[pl.ds(s * ROWS_PER_TILE, ROWS_PER_TILE)], stage_v)
        pltpu.sync_copy(stage_v, out_hbm.at[c, pl.ds(s * ROWS_PER_TILE, ROWS_PER_TILE)])

    return deg_kernel


def _make_prop_kernel(e_rows, d):
    rpt = e_rows // NW
    ncol16 = d // 16
    mesh = plsc.VectorSubcoreMesh(core_axis_name="c", subcore_axis_name="s")

    @functools.partial(
        pl.kernel,
        mesh=mesh,
        out_type=jax.ShapeDtypeStruct((NC, ACC_ROWS, d), jnp.float32),
        scratch_types=[
            pltpu.VMEM((rpt + 2, CHUNK), jnp.int32),
            pltpu.VMEM((rpt, CHUNK), jnp.int32),
            pltpu.VMEM((CHUNK, d), jnp.float32),
            pltpu.VMEM((CHUNK, d), jnp.float32),
            pltpu.VMEM((ROWS_PER_TILE, d), jnp.float32),
            pltpu.VMEM_SHARED((ACC_ROWS, d), jnp.float32),
            pltpu.SemaphoreType.DMA,
            pltpu.SemaphoreType.DMA,
        ],
        compiler_params=pltpu.CompilerParams(use_tc_tiling_on_sc=False),
    )
    def prop_kernel(src_hbm, dst_hbm, g_hbm, out_hbm,
                    src_v, dst_v, rows0_v, rows1_v, stage_v, acc_sh,
                    sem0, sem1):
        c = lax.axis_index("c")
        s = lax.axis_index("s")
        t = c * NS + s
        pltpu.sync_copy(src_hbm.at[pl.ds(t * rpt, rpt)], src_v.at[pl.ds(0, rpt)])
        pltpu.sync_copy(dst_hbm.at[pl.ds(t * rpt, rpt)], dst_v)
        zero_i = jnp.zeros((16,), jnp.int32)
        for u in range(CHUNK // 16):
            src_v[rpt, pl.ds(u * 16, 16)] = zero_i
            src_v[rpt + 1, pl.ds(u * 16, 16)] = zero_i
        _fill_zero(stage_v, ROWS_PER_TILE, ncol16)
        pltpu.sync_copy(stage_v, acc_sh.at[pl.ds(s * ROWS_PER_TILE, ROWS_PER_TILE)])
        plsc.subcore_barrier()

        def body(j, _):
            pltpu.async_copy(g_hbm.at[src_v.at[j]], rows0_v, sem0).wait()
            pltpu.sync_copy(rows0_v, acc_sh.at[dst_v.at[j]], add=True)
            return 0

        lax.fori_loop(0, rpt, body, 0)
        plsc.subcore_barrier()
        pltpu.sync_copy(acc_sh.at[pl.ds(s * ROWS_PER_TILE, ROWS_PER_TILE)], stage_v)
        pltpu.sync_copy(stage_v, out_hbm.at[c, pl.ds(s * ROWS_PER_TILE, ROWS_PER_TILE)])

    return prop_kernel


_BN = 2000


def _l1_body(x_ref, w_ref, degp_ref, g1_ref, dis_ref):
    h = jnp.dot(x_ref[...], w_ref[...], preferred_element_type=jnp.float32)
    deg = degp_ref[0] + degp_ref[1] + 1.0
    dis = lax.rsqrt(deg)
    dis_ref[...] = dis
    g1_ref[...] = h * dis[:, 0:1]


def _mid_body(p_ref, g1_ref, dis_ref, b1_ref, g2_ref):
    d = dis_ref[:, 0:1]
    z = jnp.maximum(d * (p_ref[0] + p_ref[1] + g1_ref[...]) + b1_ref[...], 0.0)
    g2_ref[...] = d * z


def _out_body(p_ref, g2_ref, dis_ref, w_ref, b_ref, o_ref):
    agg = dis_ref[:, 0:1] * (p_ref[0] + p_ref[1] + g2_ref[...])
    o_ref[...] = (
        jnp.dot(agg, w_ref[...], preferred_element_type=jnp.float32) + b_ref[...]
    )


def kernel(x, edge_index, W1, b1, W2, b2):
    n, d_in = x.shape
    d_hid = W1.shape[1]
    d_out = W2.shape[1]
    e = edge_index.shape[1]

    ei = edge_index.astype(jnp.int32)
    e_pad = ((e + NW * CHUNK - 1) // (NW * CHUNK)) * (NW * CHUNK)
    pad = e_pad - e
    pad_iota = jnp.arange(pad, dtype=jnp.int32)
    src = jnp.concatenate([ei[0], pad_iota % n])
    dst = jnp.concatenate([ei[1], n + pad_iota % (ACC_ROWS - n)])
    e_rows = e_pad // CHUNK
    src2d = src.reshape(e_rows, CHUNK)
    dst2d = dst.reshape(e_rows, CHUNK)

    deg_call = _make_deg_kernel(e_rows)
    prop_call = _make_prop_kernel(e_rows, d_hid)

    grid = (n // _BN,)
    spec_rows = lambda w: pl.BlockSpec((_BN, w), lambda i: (i, 0))
    spec_pair = lambda w: pl.BlockSpec((2, _BN, w), lambda i: (0, i, 0))
    spec_full = lambda a, b: pl.BlockSpec((a, b), lambda i: (0, 0))

    degp = deg_call(dst2d)[:, :n, :]

    g1, dis = pl.pallas_call(
        _l1_body,
        grid=grid,
        in_specs=[spec_rows(d_in), spec_full(d_in, d_hid), spec_pair(16)],
        out_specs=[spec_rows(d_hid), spec_rows(16)],
        out_shape=[
            jax.ShapeDtypeStruct((n, d_hid), jnp.float32),
            jax.ShapeDtypeStruct((n, 16), jnp.float32),
        ],
    )(x, W1, degp)

    p1 = prop_call(src2d, dst2d, g1)[:, :n, :]

    g2 = pl.pallas_call(
        _mid_body,
        grid=grid,
        in_specs=[spec_pair(d_hid), spec_rows(d_hid), spec_rows(16),
                  spec_full(1, d_hid)],
        out_specs=spec_rows(d_hid),
        out_shape=jax.ShapeDtypeStruct((n, d_hid), jnp.float32),
    )(p1, g1, dis, b1.reshape(1, d_hid))

    p2 = prop_call(src2d, dst2d, g2)[:, :n, :]

    out = pl.pallas_call(
        _out_body,
        grid=grid,
        in_specs=[spec_pair(d_hid), spec_rows(d_hid), spec_rows(16),
                  spec_full(d_hid, d_out), spec_full(1, d_out)],
        out_specs=spec_rows(d_out),
        out_shape=jax.ShapeDtypeStruct((n, d_out), jnp.float32),
    )(p2, g2, dis, W2, b2.reshape(1, d_out))

    return out


# speedup vs baseline: 2.2824x; 1.1181x over previous
import functools

import jax
import jax.numpy as jnp
from jax import lax
from jax.experimental import pallas as pl
from jax.experimental.pallas import tpu as pltpu
from jax.experimental.pallas import tpu_sc as plsc

N_NODES_C = 10000

NC = 2
NS = 16
NW = NC * NS
CHUNK = 128

ACC_ROWS = (N_NODES_C // 128 + 1) * 128
ROWS_PER_TILE = ACC_ROWS // NS


def _fill_zero(ref, n_rows, n_col16):
    zero = jnp.zeros((16,), jnp.float32)
    rpi = max(1, 8 // n_col16)

    def body(k, _):
        for ur in range(rpi):
            for uc in range(n_col16):
                ref[k * rpi + ur, pl.ds(uc * 16, 16)] = zero
        return 0

    lax.fori_loop(0, n_rows // rpi, body, 0)


def _fill_ones(ref, n_rows):
    one = jnp.ones((16,), jnp.float32)

    def body(k, _):
        for u in range(8):
            ref[k * 8 + u, :] = one
        return 0

    lax.fori_loop(0, n_rows // 8, body, 0)


def _make_deg_kernel(e_rows):
    rpt = e_rows // NW
    mesh = plsc.VectorSubcoreMesh(core_axis_name="c", subcore_axis_name="s")

    @functools.partial(
        pl.kernel,
        mesh=mesh,
        out_type=jax.ShapeDtypeStruct((NC, ACC_ROWS, 16), jnp.float32),
        scratch_types=[
            pltpu.VMEM((rpt, CHUNK), jnp.int32),
            pltpu.VMEM((CHUNK, 16), jnp.float32),
            pltpu.VMEM((ROWS_PER_TILE, 16), jnp.float32),
            pltpu.VMEM_SHARED((ACC_ROWS, 16), jnp.float32),
        ],
        compiler_params=pltpu.CompilerParams(use_tc_tiling_on_sc=False),
    )
    def deg_kernel(dst_hbm, out_hbm, dst_v, ones_v, stage_v, acc_sh):
        c = lax.axis_index("c")
        s = lax.axis_index("s")
        t = c * NS + s
        pltpu.sync_copy(dst_hbm.at[pl.ds(t * rpt, rpt)], dst_v)
        _fill_ones(ones_v, CHUNK)
        _fill_zero(stage_v, ROWS_PER_TILE, 1)
        pltpu.sync_copy(stage_v, acc_sh.at[pl.ds(s * ROWS_PER_TILE, ROWS_PER_TILE)])
        plsc.subcore_barrier()

        def body(j, _):
            pltpu.sync_copy(ones_v, acc_sh.at[dst_v.at[j]], add=True)
            return 0

        lax.fori_loop(0, rpt, body, 0)
        plsc.subcore_barrier()
        pltpu.sync_copy(acc_sh.at[pl.ds(s * ROWS_PER_TILE, ROWS_PER_TILE)], stage_v)
        pltpu.sync_copy(stage_v, out_hbm.at[c, pl.ds(s * ROWS_PER_TILE, ROWS_PER_TILE)])

    return deg_kernel


def _make_prop_kernel(e_rows, d):
    rpt = e_rows // NW
    ncol16 = d // 16
    mesh = plsc.VectorSubcoreMesh(core_axis_name="c", subcore_axis_name="s")

    @functools.partial(
        pl.kernel,
        mesh=mesh,
        out_type=jax.ShapeDtypeStruct((NC, ACC_ROWS, d), jnp.float32),
        scratch_types=[
            pltpu.VMEM((rpt + 2, CHUNK), jnp.int32),
            pltpu.VMEM((rpt, CHUNK), jnp.int32),
            pltpu.VMEM((CHUNK, d), jnp.float32),
            pltpu.VMEM((CHUNK, d), jnp.float32),
            pltpu.VMEM((ROWS_PER_TILE, d), jnp.float32),
            pltpu.VMEM_SHARED((ACC_ROWS, d), jnp.float32),
            pltpu.SemaphoreType.DMA,
            pltpu.SemaphoreType.DMA,
        ],
        compiler_params=pltpu.CompilerParams(use_tc_tiling_on_sc=False),
    )
    def prop_kernel(src_hbm, dst_hbm, g_hbm, out_hbm,
                    src_v, dst_v, rows0_v, rows1_v, stage_v, acc_sh,
                    sem0, sem1):
        c = lax.axis_index("c")
        s = lax.axis_index("s")
        t = c * NS + s
        pltpu.sync_copy(src_hbm.at[pl.ds(t * rpt, rpt)], src_v.at[pl.ds(0, rpt)])
        pltpu.sync_copy(dst_hbm.at[pl.ds(t * rpt, rpt)], dst_v)
        zero_i = jnp.zeros((16,), jnp.int32)
        for u in range(CHUNK // 16):
            src_v[rpt, pl.ds(u * 16, 16)] = zero_i
            src_v[rpt + 1, pl.ds(u * 16, 16)] = zero_i
        _fill_zero(stage_v, ROWS_PER_TILE, ncol16)
        pltpu.sync_copy(stage_v, acc_sh.at[pl.ds(s * ROWS_PER_TILE, ROWS_PER_TILE)])
        plsc.subcore_barrier()

        def body(g, _):
            j = g * 2
            h0 = pltpu.async_copy(g_hbm.at[src_v.at[j]], rows0_v, sem0)
            h1 = pltpu.async_copy(g_hbm.at[src_v.at[j + 1]], rows1_v, sem1)
            h0.wait()
            pltpu.sync_copy(rows0_v, acc_sh.at[dst_v.at[j]], add=True)
            h1.wait()
            pltpu.sync_copy(rows1_v, acc_sh.at[dst_v.at[j + 1]], add=True)
            return 0

        lax.fori_loop(0, rpt // 2, body, 0)
        plsc.subcore_barrier()
        pltpu.sync_copy(acc_sh.at[pl.ds(s * ROWS_PER_TILE, ROWS_PER_TILE)], stage_v)
        pltpu.sync_copy(stage_v, out_hbm.at[c, pl.ds(s * ROWS_PER_TILE, ROWS_PER_TILE)])

    return prop_kernel


_BN = 2000


def _l1_body(x_ref, w_ref, degp_ref, g1_ref, dis_ref):
    h = jnp.dot(x_ref[...], w_ref[...], preferred_element_type=jnp.float32)
    deg = degp_ref[0] + degp_ref[1] + 1.0
    dis = lax.rsqrt(deg)
    dis_ref[...] = dis
    g1_ref[...] = h * dis[:, 0:1]


def _mid_body(p_ref, g1_ref, dis_ref, b1_ref, g2_ref):
    d = dis_ref[:, 0:1]
    z = jnp.maximum(d * (p_ref[0] + p_ref[1] + g1_ref[...]) + b1_ref[...], 0.0)
    g2_ref[...] = d * z


def _out_body(p_ref, g2_ref, dis_ref, w_ref, b_ref, o_ref):
    agg = dis_ref[:, 0:1] * (p_ref[0] + p_ref[1] + g2_ref[...])
    o_ref[...] = (
        jnp.dot(agg, w_ref[...], preferred_element_type=jnp.float32) + b_ref[...]
    )


def kernel(x, edge_index, W1, b1, W2, b2):
    n, d_in = x.shape
    d_hid = W1.shape[1]
    d_out = W2.shape[1]
    e = edge_index.shape[1]

    ei = edge_index.astype(jnp.int32)
    e_pad = ((e + NW * CHUNK - 1) // (NW * CHUNK)) * (NW * CHUNK)
    pad = e_pad - e
    pad_iota = jnp.arange(pad, dtype=jnp.int32)
    src = jnp.concatenate([ei[0], pad_iota % n])
    dst = jnp.concatenate([ei[1], n + pad_iota % (ACC_ROWS - n)])
    e_rows = e_pad // CHUNK
    src2d = src.reshape(e_rows, CHUNK)
    dst2d = dst.reshape(e_rows, CHUNK)

    deg_call = _make_deg_kernel(e_rows)
    prop_call = _make_prop_kernel(e_rows, d_hid)

    grid = (n // _BN,)
    spec_rows = lambda w: pl.BlockSpec((_BN, w), lambda i: (i, 0))
    spec_pair = lambda w: pl.BlockSpec((2, _BN, w), lambda i: (0, i, 0))
    spec_full = lambda a, b: pl.BlockSpec((a, b), lambda i: (0, 0))

    degp = deg_call(dst2d)[:, :n, :]

    g1, dis = pl.pallas_call(
        _l1_body,
        grid=grid,
        in_specs=[spec_rows(d_in), spec_full(d_in, d_hid), spec_pair(16)],
        out_specs=[spec_rows(d_hid), spec_rows(16)],
        out_shape=[
            jax.ShapeDtypeStruct((n, d_hid), jnp.float32),
            jax.ShapeDtypeStruct((n, 16), jnp.float32),
        ],
    )(x, W1, degp)

    p1 = prop_call(src2d, dst2d, g1)[:, :n, :]

    g2 = pl.pallas_call(
        _mid_body,
        grid=grid,
        in_specs=[spec_pair(d_hid), spec_rows(d_hid), spec_rows(16),
                  spec_full(1, d_hid)],
        out_specs=spec_rows(d_hid),
        out_shape=jax.ShapeDtypeStruct((n, d_hid), jnp.float32),
    )(p1, g1, dis, b1.reshape(1, d_hid))

    p2 = prop_call(src2d, dst2d, g2)[:, :n, :]

    out = pl.pallas_call(
        _out_body,
        grid=grid,
        in_specs=[spec_pair(d_hid), spec_rows(d_hid), spec_rows(16),
                  spec_full(d_hid, d_out), spec_full(1, d_out)],
        out_specs=spec_rows(d_out),
        out_shape=jax.ShapeDtypeStruct((n, d_out), jnp.float32),
    )(p2, g2, dis, W2, b2.reshape(1, d_out))

    return out


# speedup vs baseline: 2.3659x; 1.0366x over previous
import functools

import jax
import jax.numpy as jnp
from jax import lax
from jax.experimental import pallas as pl
from jax.experimental.pallas import tpu as pltpu
from jax.experimental.pallas import tpu_sc as plsc

N_NODES_C = 10000

NC = 2
NS = 16
NW = NC * NS
CHUNK = 128

ACC_ROWS = (N_NODES_C // 128 + 1) * 128
ROWS_PER_TILE = ACC_ROWS // NS


def _fill_zero(ref, n_rows, n_col16):
    zero = jnp.zeros((16,), jnp.float32)
    rpi = max(1, 8 // n_col16)

    def body(k, _):
        for ur in range(rpi):
            for uc in range(n_col16):
                ref[k * rpi + ur, pl.ds(uc * 16, 16)] = zero
        return 0

    lax.fori_loop(0, n_rows // rpi, body, 0)


def _fill_ones(ref, n_rows):
    one = jnp.ones((16,), jnp.float32)

    def body(k, _):
        for u in range(8):
            ref[k * 8 + u, :] = one
        return 0

    lax.fori_loop(0, n_rows // 8, body, 0)


def _make_deg_kernel(e_rows):
    rpt = e_rows // NW
    mesh = plsc.VectorSubcoreMesh(core_axis_name="c", subcore_axis_name="s")

    @functools.partial(
        pl.kernel,
        mesh=mesh,
        out_type=jax.ShapeDtypeStruct((NC, ACC_ROWS, 16), jnp.float32),
        scratch_types=[
            pltpu.VMEM((rpt, CHUNK), jnp.int32),
            pltpu.VMEM((CHUNK, 16), jnp.float32),
            pltpu.VMEM((ROWS_PER_TILE, 16), jnp.float32),
            pltpu.VMEM_SHARED((ACC_ROWS, 16), jnp.float32),
        ],
        compiler_params=pltpu.CompilerParams(use_tc_tiling_on_sc=False),
    )
    def deg_kernel(dst_hbm, out_hbm, dst_v, ones_v, stage_v, acc_sh):
        c = lax.axis_index("c")
        s = lax.axis_index("s")
        t = c * NS + s
        pltpu.sync_copy(dst_hbm.at[pl.ds(t * rpt, rpt)], dst_v)
        _fill_ones(ones_v, CHUNK)
        _fill_zero(stage_v, ROWS_PER_TILE, 1)
        pltpu.sync_copy(stage_v, acc_sh.at[pl.ds(s * ROWS_PER_TILE, ROWS_PER_TILE)])
        plsc.subcore_barrier()

        def body(j, _):
            pltpu.sync_copy(ones_v, acc_sh.at[dst_v.at[j]], add=True)
            return 0

        lax.fori_loop(0, rpt, body, 0)
        plsc.subcore_barrier()
        pltpu.sync_copy(acc_sh.at[pl.ds(s * ROWS_PER_TILE, ROWS_PER_TILE)], stage_v)
        pltpu.sync_copy(stage_v, out_hbm.at[c, pl.ds(s * ROWS_PER_TILE, ROWS_PER_TILE)])

    return deg_kernel


def _make_prop_kernel(e_rows, d):
    rpt = e_rows // NW
    ncol16 = d // 16
    mesh = plsc.VectorSubcoreMesh(core_axis_name="c", subcore_axis_name="s")

    @functools.partial(
        pl.kernel,
        mesh=mesh,
        out_type=jax.ShapeDtypeStruct((NC, ACC_ROWS, d), jnp.float32),
        scratch_types=[
            pltpu.VMEM((rpt + 2, CHUNK), jnp.int32),
            pltpu.VMEM((rpt, CHUNK), jnp.int32),
            pltpu.VMEM((CHUNK, d), jnp.float32),
            pltpu.VMEM((CHUNK, d), jnp.float32),
            pltpu.VMEM((CHUNK, d), jnp.float32),
            pltpu.VMEM((CHUNK, d), jnp.float32),
            pltpu.VMEM((ROWS_PER_TILE, d), jnp.float32),
            pltpu.VMEM_SHARED((ACC_ROWS, d), jnp.float32),
            pltpu.SemaphoreType.DMA,
            pltpu.SemaphoreType.DMA,
            pltpu.SemaphoreType.DMA,
            pltpu.SemaphoreType.DMA,
        ],
        compiler_params=pltpu.CompilerParams(use_tc_tiling_on_sc=False),
    )
    def prop_kernel(src_hbm, dst_hbm, g_hbm, out_hbm,
                    src_v, dst_v, rows0_v, rows1_v, rows2_v, rows3_v,
                    stage_v, acc_sh, sem0, sem1, sem2, sem3):
        c = lax.axis_index("c")
        s = lax.axis_index("s")
        t = c * NS + s
        pltpu.sync_copy(src_hbm.at[pl.ds(t * rpt, rpt)], src_v.at[pl.ds(0, rpt)])
        pltpu.sync_copy(dst_hbm.at[pl.ds(t * rpt, rpt)], dst_v)
        zero_i = jnp.zeros((16,), jnp.int32)
        for u in range(CHUNK // 16):
            src_v[rpt, pl.ds(u * 16, 16)] = zero_i
            src_v[rpt + 1, pl.ds(u * 16, 16)] = zero_i
        _fill_zero(stage_v, ROWS_PER_TILE, ncol16)
        pltpu.sync_copy(stage_v, acc_sh.at[pl.ds(s * ROWS_PER_TILE, ROWS_PER_TILE)])
        plsc.subcore_barrier()

        bufs = (rows0_v, rows1_v, rows2_v, rows3_v)
        sems = (sem0, sem1, sem2, sem3)

        def body(g, _):
            j = g * 4
            hs = [pltpu.async_copy(g_hbm.at[src_v.at[j + b]], bufs[b], sems[b])
                  for b in range(4)]
            for b in range(4):
                hs[b].wait()
                pltpu.sync_copy(bufs[b], acc_sh.at[dst_v.at[j + b]], add=True)
            return 0

        lax.fori_loop(0, rpt // 4, body, 0)
        plsc.subcore_barrier()
        pltpu.sync_copy(acc_sh.at[pl.ds(s * ROWS_PER_TILE, ROWS_PER_TILE)], stage_v)
        pltpu.sync_copy(stage_v, out_hbm.at[c, pl.ds(s * ROWS_PER_TILE, ROWS_PER_TILE)])

    return prop_kernel


_BN = 2000


def _l1_body(x_ref, w_ref, degp_ref, g1_ref, dis_ref):
    h = jnp.dot(x_ref[...], w_ref[...], preferred_element_type=jnp.float32)
    deg = degp_ref[0] + degp_ref[1] + 1.0
    dis = lax.rsqrt(deg)
    dis_ref[...] = dis
    g1_ref[...] = h * dis[:, 0:1]


def _mid_body(p_ref, g1_ref, dis_ref, b1_ref, g2_ref):
    d = dis_ref[:, 0:1]
    z = jnp.maximum(d * (p_ref[0] + p_ref[1] + g1_ref[...]) + b1_ref[...], 0.0)
    g2_ref[...] = d * z


def _out_body(p_ref, g2_ref, dis_ref, w_ref, b_ref, o_ref):
    agg = dis_ref[:, 0:1] * (p_ref[0] + p_ref[1] + g2_ref[...])
    o_ref[...] = (
        jnp.dot(agg, w_ref[...], preferred_element_type=jnp.float32) + b_ref[...]
    )


def kernel(x, edge_index, W1, b1, W2, b2):
    n, d_in = x.shape
    d_hid = W1.shape[1]
    d_out = W2.shape[1]
    e = edge_index.shape[1]

    ei = edge_index.astype(jnp.int32)
    e_pad = ((e + NW * CHUNK - 1) // (NW * CHUNK)) * (NW * CHUNK)
    pad = e_pad - e
    pad_iota = jnp.arange(pad, dtype=jnp.int32)
    src = jnp.concatenate([ei[0], pad_iota % n])
    dst = jnp.concatenate([ei[1], n + pad_iota % (ACC_ROWS - n)])
    e_rows = e_pad // CHUNK
    src2d = src.reshape(e_rows, CHUNK)
    dst2d = dst.reshape(e_rows, CHUNK)

    deg_call = _make_deg_kernel(e_rows)
    prop_call = _make_prop_kernel(e_rows, d_hid)

    grid = (n // _BN,)
    spec_rows = lambda w: pl.BlockSpec((_BN, w), lambda i: (i, 0))
    spec_pair = lambda w: pl.BlockSpec((2, _BN, w), lambda i: (0, i, 0))
    spec_full = lambda a, b: pl.BlockSpec((a, b), lambda i: (0, 0))

    degp = deg_call(dst2d)[:, :n, :]

    g1, dis = pl.pallas_call(
        _l1_body,
        grid=grid,
        in_specs=[spec_rows(d_in), spec_full(d_in, d_hid), spec_pair(16)],
        out_specs=[spec_rows(d_hid), spec_rows(16)],
        out_shape=[
            jax.ShapeDtypeStruct((n, d_hid), jnp.float32),
            jax.ShapeDtypeStruct((n, 16), jnp.float32),
        ],
    )(x, W1, degp)

    p1 = prop_call(src2d, dst2d, g1)[:, :n, :]

    g2 = pl.pallas_call(
        _mid_body,
        grid=grid,
        in_specs=[spec_pair(d_hid), spec_rows(d_hid), spec_rows(16),
                  spec_full(1, d_hid)],
        out_specs=spec_rows(d_hid),
        out_shape=jax.ShapeDtypeStruct((n, d_hid), jnp.float32),
    )(p1, g1, dis, b1.reshape(1, d_hid))

    p2 = prop_call(src2d, dst2d, g2)[:, :n, :]

    out = pl.pallas_call(
        _out_body,
        grid=grid,
        in_specs=[spec_pair(d_hid), spec_rows(d_hid), spec_rows(16),
                  spec_full(d_hid, d_out), spec_full(1, d_out)],
        out_specs=spec_rows(d_out),
        out_shape=jax.ShapeDtypeStruct((n, d_out), jnp.float32),
    )(p2, g2, dis, W2, b2.reshape(1, d_out))

    return out


# speedup vs baseline: 2.4058x; 1.0169x over previous
import functools

import jax
import jax.numpy as jnp
from jax import lax
from jax.experimental import pallas as pl
from jax.experimental.pallas import tpu as pltpu
from jax.experimental.pallas import tpu_sc as plsc

N_NODES_C = 10000

NC = 2
NS = 16
NW = NC * NS
CHUNK = 128

ACC_ROWS = (N_NODES_C // 128 + 1) * 128
ROWS_PER_TILE = ACC_ROWS // NS


def _fill_zero(ref, n_rows, n_col16):
    zero = jnp.zeros((16,), jnp.float32)
    rpi = max(1, 8 // n_col16)

    def body(k, _):
        for ur in range(rpi):
            for uc in range(n_col16):
                ref[k * rpi + ur, pl.ds(uc * 16, 16)] = zero
        return 0

    lax.fori_loop(0, n_rows // rpi, body, 0)


def _fill_ones(ref, n_rows):
    one = jnp.ones((16,), jnp.float32)

    def body(k, _):
        for u in range(8):
            ref[k * 8 + u, :] = one
        return 0

    lax.fori_loop(0, n_rows // 8, body, 0)


def _make_deg_kernel(e_rows):
    rpt = e_rows // NW
    mesh = plsc.VectorSubcoreMesh(core_axis_name="c", subcore_axis_name="s")

    @functools.partial(
        pl.kernel,
        mesh=mesh,
        out_type=jax.ShapeDtypeStruct((NC, ACC_ROWS, 16), jnp.float32),
        scratch_types=[
            pltpu.VMEM((rpt, CHUNK), jnp.int32),
            pltpu.VMEM((CHUNK, 16), jnp.float32),
            pltpu.VMEM((ROWS_PER_TILE, 16), jnp.float32),
            pltpu.VMEM_SHARED((ACC_ROWS, 16), jnp.float32),
        ],
        compiler_params=pltpu.CompilerParams(use_tc_tiling_on_sc=False),
    )
    def deg_kernel(dst_hbm, out_hbm, dst_v, ones_v, stage_v, acc_sh):
        c = lax.axis_index("c")
        s = lax.axis_index("s")
        t = c * NS + s
        pltpu.sync_copy(dst_hbm.at[pl.ds(t * rpt, rpt)], dst_v)
        _fill_ones(ones_v, CHUNK)
        _fill_zero(stage_v, ROWS_PER_TILE, 1)
        pltpu.sync_copy(stage_v, acc_sh.at[pl.ds(s * ROWS_PER_TILE, ROWS_PER_TILE)])
        plsc.subcore_barrier()

        def body(j, _):
            pltpu.sync_copy(ones_v, acc_sh.at[dst_v.at[j]], add=True)
            return 0

        lax.fori_loop(0, rpt, body, 0)
        plsc.subcore_barrier()
        pltpu.sync_copy(acc_sh.at[pl.ds(s * ROWS_PER_TILE, ROWS_PER_TILE)], stage_v)
        pltpu.sync_copy(stage_v, out_hbm.at[c, pl.ds(s * ROWS_PER_TILE, ROWS_PER_TILE)])

    return deg_kernel


def _make_prop_kernel(e_rows, d):
    rpt = e_rows // NW
    ncol16 = d // 16
    mesh = plsc.VectorSubcoreMesh(core_axis_name="c", subcore_axis_name="s")

    @functools.partial(
        pl.kernel,
        mesh=mesh,
        out_type=jax.ShapeDtypeStruct((NC, ACC_ROWS, d), jnp.float32),
        scratch_types=[
            pltpu.VMEM((rpt + 2, CHUNK), jnp.int32),
            pltpu.VMEM((rpt, CHUNK), jnp.int32),
            pltpu.VMEM((CHUNK, d), jnp.float32),
            pltpu.VMEM((CHUNK, d), jnp.float32),
            pltpu.VMEM((CHUNK, d), jnp.float32),
            pltpu.VMEM((CHUNK, d), jnp.float32),
            pltpu.VMEM((ROWS_PER_TILE, d), jnp.float32),
            pltpu.VMEM_SHARED((ACC_ROWS, d), jnp.float32),
            pltpu.SemaphoreType.DMA,
            pltpu.SemaphoreType.DMA,
            pltpu.SemaphoreType.DMA,
            pltpu.SemaphoreType.DMA,
            pltpu.SemaphoreType.DMA,
            pltpu.SemaphoreType.DMA,
            pltpu.SemaphoreType.DMA,
            pltpu.SemaphoreType.DMA,
        ],
        compiler_params=pltpu.CompilerParams(use_tc_tiling_on_sc=False),
    )
    def prop_kernel(src_hbm, dst_hbm, g_hbm, out_hbm,
                    src_v, dst_v, rows0_v, rows1_v, rows2_v, rows3_v,
                    stage_v, acc_sh, sem0, sem1, sem2, sem3,
                    ssem0, ssem1, ssem2, ssem3):
        c = lax.axis_index("c")
        s = lax.axis_index("s")
        t = c * NS + s
        pltpu.sync_copy(src_hbm.at[pl.ds(t * rpt, rpt)], src_v.at[pl.ds(0, rpt)])
        pltpu.sync_copy(dst_hbm.at[pl.ds(t * rpt, rpt)], dst_v)
        zero_i = jnp.zeros((16,), jnp.int32)
        for u in range(CHUNK // 16):
            src_v[rpt, pl.ds(u * 16, 16)] = zero_i
            src_v[rpt + 1, pl.ds(u * 16, 16)] = zero_i
        _fill_zero(stage_v, ROWS_PER_TILE, ncol16)
        pltpu.sync_copy(stage_v, acc_sh.at[pl.ds(s * ROWS_PER_TILE, ROWS_PER_TILE)])
        plsc.subcore_barrier()

        bufs = (rows0_v, rows1_v, rows2_v, rows3_v)
        sems = (sem0, sem1, sem2, sem3)
        ssems = (ssem0, ssem1, ssem2, ssem3)

        def body(g, _):
            j = g * 4
            hs = [pltpu.async_copy(g_hbm.at[src_v.at[j + b]], bufs[b], sems[b])
                  for b in range(4)]
            ss = []
            for b in range(4):
                hs[b].wait()
                ss.append(pltpu.async_copy(
                    bufs[b], acc_sh.at[dst_v.at[j + b]], ssems[b], add=True))
            for b in range(4):
                ss[b].wait()
            return 0

        lax.fori_loop(0, rpt // 4, body, 0)
        plsc.subcore_barrier()
        pltpu.sync_copy(acc_sh.at[pl.ds(s * ROWS_PER_TILE, ROWS_PER_TILE)], stage_v)
        pltpu.sync_copy(stage_v, out_hbm.at[c, pl.ds(s * ROWS_PER_TILE, ROWS_PER_TILE)])

    return prop_kernel


_BN = 2000


def _l1_body(x_ref, w_ref, degp_ref, g1_ref, dis_ref):
    h = jnp.dot(x_ref[...], w_ref[...], preferred_element_type=jnp.float32)
    deg = degp_ref[0] + degp_ref[1] + 1.0
    dis = lax.rsqrt(deg)
    dis_ref[...] = dis
    g1_ref[...] = h * dis[:, 0:1]


def _mid_body(p_ref, g1_ref, dis_ref, b1_ref, g2_ref):
    d = dis_ref[:, 0:1]
    z = jnp.maximum(d * (p_ref[0] + p_ref[1] + g1_ref[...]) + b1_ref[...], 0.0)
    g2_ref[...] = d * z


def _out_body(p_ref, g2_ref, dis_ref, w_ref, b_ref, o_ref):
    agg = dis_ref[:, 0:1] * (p_ref[0] + p_ref[1] + g2_ref[...])
    o_ref[...] = (
        jnp.dot(agg, w_ref[...], preferred_element_type=jnp.float32) + b_ref[...]
    )


def kernel(x, edge_index, W1, b1, W2, b2):
    n, d_in = x.shape
    d_hid = W1.shape[1]
    d_out = W2.shape[1]
    e = edge_index.shape[1]

    ei = edge_index.astype(jnp.int32)
    e_pad = ((e + NW * CHUNK - 1) // (NW * CHUNK)) * (NW * CHUNK)
    pad = e_pad - e
    pad_iota = jnp.arange(pad, dtype=jnp.int32)
    src = jnp.concatenate([ei[0], pad_iota % n])
    dst = jnp.concatenate([ei[1], n + pad_iota % (ACC_ROWS - n)])
    e_rows = e_pad // CHUNK
    src2d = src.reshape(e_rows, CHUNK)
    dst2d = dst.reshape(e_rows, CHUNK)

    deg_call = _make_deg_kernel(e_rows)
    prop_call = _make_prop_kernel(e_rows, d_hid)

    grid = (n // _BN,)
    spec_rows = lambda w: pl.BlockSpec((_BN, w), lambda i: (i, 0))
    spec_pair = lambda w: pl.BlockSpec((2, _BN, w), lambda i: (0, i, 0))
    spec_full = lambda a, b: pl.BlockSpec((a, b), lambda i: (0, 0))

    degp = deg_call(dst2d)[:, :n, :]

    g1, dis = pl.pallas_call(
        _l1_body,
        grid=grid,
        in_specs=[spec_rows(d_in), spec_full(d_in, d_hid), spec_pair(16)],
        out_specs=[spec_rows(d_hid), spec_rows(16)],
        out_shape=[
            jax.ShapeDtypeStruct((n, d_hid), jnp.float32),
            jax.ShapeDtypeStruct((n, 16), jnp.float32),
        ],
    )(x, W1, degp)

    p1 = prop_call(src2d, dst2d, g1)[:, :n, :]

    g2 = pl.pallas_call(
        _mid_body,
        grid=grid,
        in_specs=[spec_pair(d_hid), spec_rows(d_hid), spec_rows(16),
                  spec_full(1, d_hid)],
        out_specs=spec_rows(d_hid),
        out_shape=jax.ShapeDtypeStruct((n, d_hid), jnp.float32),
    )(p1, g1, dis, b1.reshape(1, d_hid))

    p2 = prop_call(src2d, dst2d, g2)[:, :n, :]

    out = pl.pallas_call(
        _out_body,
        grid=grid,
        in_specs=[spec_pair(d_hid), spec_rows(d_hid), spec_rows(16),
                  spec_full(d_hid, d_out), spec_full(1, d_out)],
        out_specs=spec_rows(d_out),
        out_shape=jax.ShapeDtypeStruct((n, d_out), jnp.float32),
    )(p2, g2, dis, W2, b2.reshape(1, d_out))

    return out


# speedup vs baseline: 2.4059x; 1.0001x over previous
import functools

import jax
import jax.numpy as jnp
from jax import lax
from jax.experimental import pallas as pl
from jax.experimental.pallas import tpu as pltpu
from jax.experimental.pallas import tpu_sc as plsc

N_NODES_C = 10000

NC = 2
NS = 16
NW = NC * NS
CHUNK = 128

ACC_ROWS = (N_NODES_C // 128 + 1) * 128
ROWS_PER_TILE = ACC_ROWS // NS


def _fill_zero(ref, n_rows, n_col16):
    zero = jnp.zeros((16,), jnp.float32)
    rpi = max(1, 8 // n_col16)

    def body(k, _):
        for ur in range(rpi):
            for uc in range(n_col16):
                ref[k * rpi + ur, pl.ds(uc * 16, 16)] = zero
        return 0

    lax.fori_loop(0, n_rows // rpi, body, 0)


def _fill_ones(ref, n_rows):
    one = jnp.ones((16,), jnp.float32)

    def body(k, _):
        for u in range(8):
            ref[k * 8 + u, :] = one
        return 0

    lax.fori_loop(0, n_rows // 8, body, 0)


def _make_deg_kernel(e_rows):
    rpt = e_rows // NW
    mesh = plsc.VectorSubcoreMesh(core_axis_name="c", subcore_axis_name="s")

    @functools.partial(
        pl.kernel,
        mesh=mesh,
        out_type=jax.ShapeDtypeStruct((NC, ACC_ROWS, 16), jnp.float32),
        scratch_types=[
            pltpu.VMEM((rpt, CHUNK), jnp.int32),
            pltpu.VMEM((CHUNK, 16), jnp.float32),
            pltpu.VMEM((ROWS_PER_TILE, 16), jnp.float32),
            pltpu.VMEM_SHARED((ACC_ROWS, 16), jnp.float32),
        ],
        compiler_params=pltpu.CompilerParams(use_tc_tiling_on_sc=False),
    )
    def deg_kernel(dst_hbm, out_hbm, dst_v, ones_v, stage_v, acc_sh):
        c = lax.axis_index("c")
        s = lax.axis_index("s")
        t = c * NS + s
        pltpu.sync_copy(dst_hbm.at[pl.ds(t * rpt, rpt)], dst_v)
        _fill_ones(ones_v, CHUNK)
        _fill_zero(stage_v, ROWS_PER_TILE, 1)
        pltpu.sync_copy(stage_v, acc_sh.at[pl.ds(s * ROWS_PER_TILE, ROWS_PER_TILE)])
        plsc.subcore_barrier()

        def body(j, _):
            pltpu.sync_copy(ones_v, acc_sh.at[dst_v.at[j]], add=True)
            return 0

        lax.fori_loop(0, rpt, body, 0)
        plsc.subcore_barrier()
        pltpu.sync_copy(acc_sh.at[pl.ds(s * ROWS_PER_TILE, ROWS_PER_TILE)], stage_v)
        pltpu.sync_copy(stage_v, out_hbm.at[c, pl.ds(s * ROWS_PER_TILE, ROWS_PER_TILE)])

    return deg_kernel


def _make_prop_kernel(e_rows, d):
    rpt = e_rows // NW
    ncol16 = d // 16
    mesh = plsc.VectorSubcoreMesh(core_axis_name="c", subcore_axis_name="s")

    @functools.partial(
        pl.kernel,
        mesh=mesh,
        out_type=jax.ShapeDtypeStruct((NC, ACC_ROWS, d), jnp.float32),
        scratch_types=[
            pltpu.VMEM((rpt, CHUNK), jnp.int32),
            pltpu.VMEM((rpt, CHUNK), jnp.int32),
            pltpu.VMEM((CHUNK, d), jnp.float32),
            pltpu.VMEM((CHUNK, d), jnp.float32),
            pltpu.VMEM((CHUNK, d), jnp.float32),
            pltpu.VMEM((CHUNK, d), jnp.float32),
            pltpu.VMEM((ROWS_PER_TILE, d), jnp.float32),
            pltpu.VMEM_SHARED((ACC_ROWS, d), jnp.float32),
            pltpu.SemaphoreType.DMA,
            pltpu.SemaphoreType.DMA,
            pltpu.SemaphoreType.DMA,
            pltpu.SemaphoreType.DMA,
            pltpu.SemaphoreType.DMA,
            pltpu.SemaphoreType.DMA,
            pltpu.SemaphoreType.DMA,
            pltpu.SemaphoreType.DMA,
        ],
        compiler_params=pltpu.CompilerParams(use_tc_tiling_on_sc=False),
    )
    def prop_kernel(src_hbm, dst_hbm, g_hbm, out_hbm,
                    src_v, dst_v, rows0_v, rows1_v, rows2_v, rows3_v,
                    stage_v, acc_sh, sem0, sem1, sem2, sem3,
                    ssem0, ssem1, ssem2, ssem3):
        c = lax.axis_index("c")
        s = lax.axis_index("s")
        t = c * NS + s
        pltpu.sync_copy(src_hbm.at[pl.ds(t * rpt, rpt)], src_v)
        pltpu.sync_copy(dst_hbm.at[pl.ds(t * rpt, rpt)], dst_v)
        _fill_zero(stage_v, ROWS_PER_TILE, ncol16)
        pltpu.sync_copy(stage_v, acc_sh.at[pl.ds(s * ROWS_PER_TILE, ROWS_PER_TILE)])
        plsc.subcore_barrier()

        bufs = (rows0_v, rows1_v, rows2_v, rows3_v)
        sems = (sem0, sem1, sem2, sem3)
        ssems = (ssem0, ssem1, ssem2, ssem3)

        def body(g, _):
            j = g * 4
            hs = [pltpu.async_copy(g_hbm.at[src_v.at[j + b]], bufs[b], sems[b])
                  for b in range(4)]
            ss = []
            for b in range(4):
                hs[b].wait()
                ss.append(pltpu.async_copy(
                    bufs[b], acc_sh.at[dst_v.at[j + b]], ssems[b], add=True))
            for b in range(4):
                ss[b].wait()
            return 0

        lax.fori_loop(0, rpt // 4, body, 0)
        plsc.subcore_barrier()
        pltpu.sync_copy(acc_sh.at[pl.ds(s * ROWS_PER_TILE, ROWS_PER_TILE)], stage_v)
        pltpu.sync_copy(stage_v, out_hbm.at[c, pl.ds(s * ROWS_PER_TILE, ROWS_PER_TILE)])

    return prop_kernel


_BN = 2000


def _l1_body(x_ref, w_ref, degp_ref, g1_ref, dis_ref):
    h = jnp.dot(x_ref[...], w_ref[...], preferred_element_type=jnp.float32)
    deg = degp_ref[0] + degp_ref[1] + 1.0
    dis = lax.rsqrt(deg)
    dis_ref[...] = dis
    g1_ref[...] = h * dis[:, 0:1]


def _mid_body(p_ref, g1_ref, dis_ref, b1_ref, g2_ref):
    d = dis_ref[:, 0:1]
    z = jnp.maximum(d * (p_ref[0] + p_ref[1] + g1_ref[...]) + b1_ref[...], 0.0)
    g2_ref[...] = d * z


def _out_body(p_ref, g2_ref, dis_ref, w_ref, b_ref, o_ref):
    agg = dis_ref[:, 0:1] * (p_ref[0] + p_ref[1] + g2_ref[...])
    o_ref[...] = (
        jnp.dot(agg, w_ref[...], preferred_element_type=jnp.float32) + b_ref[...]
    )


def kernel(x, edge_index, W1, b1, W2, b2):
    n, d_in = x.shape
    d_hid = W1.shape[1]
    d_out = W2.shape[1]
    e = edge_index.shape[1]

    ei = edge_index.astype(jnp.int32)
    e_pad = ((e + NW * CHUNK - 1) // (NW * CHUNK)) * (NW * CHUNK)
    pad = e_pad - e
    pad_iota = jnp.arange(pad, dtype=jnp.int32)
    src = jnp.concatenate([ei[0], pad_iota % n])
    dst = jnp.concatenate([ei[1], n + pad_iota % (ACC_ROWS - n)])
    e_rows = e_pad // CHUNK
    src2d = src.reshape(e_rows, CHUNK)
    dst2d = dst.reshape(e_rows, CHUNK)

    deg_call = _make_deg_kernel(e_rows)
    prop_call = _make_prop_kernel(e_rows, d_hid)

    grid = (n // _BN,)
    spec_rows = lambda w: pl.BlockSpec((_BN, w), lambda i: (i, 0))
    spec_pair = lambda w: pl.BlockSpec((2, _BN, w), lambda i: (0, i, 0))
    spec_full = lambda a, b: pl.BlockSpec((a, b), lambda i: (0, 0))

    degp = deg_call(dst2d)[:, :n, :]

    g1, dis = pl.pallas_call(
        _l1_body,
        grid=grid,
        in_specs=[spec_rows(d_in), spec_full(d_in, d_hid), spec_pair(16)],
        out_specs=[spec_rows(d_hid), spec_rows(16)],
        out_shape=[
            jax.ShapeDtypeStruct((n, d_hid), jnp.float32),
            jax.ShapeDtypeStruct((n, 16), jnp.float32),
        ],
    )(x, W1, degp)

    p1 = prop_call(src2d, dst2d, g1)[:, :n, :]

    g2 = pl.pallas_call(
        _mid_body,
        grid=grid,
        in_specs=[spec_pair(d_hid), spec_rows(d_hid), spec_rows(16),
                  spec_full(1, d_hid)],
        out_specs=spec_rows(d_hid),
        out_shape=jax.ShapeDtypeStruct((n, d_hid), jnp.float32),
    )(p1, g1, dis, b1.reshape(1, d_hid))

    p2 = prop_call(src2d, dst2d, g2)[:, :n, :]

    out = pl.pallas_call(
        _out_body,
        grid=grid,
        in_specs=[spec_pair(d_hid), spec_rows(d_hid), spec_rows(16),
                  spec_full(d_hid, d_out), spec_full(1, d_out)],
        out_specs=spec_rows(d_out),
        out_shape=jax.ShapeDtypeStruct((n, d_out), jnp.float32),
    )(p2, g2, dis, W2, b2.reshape(1, d_out))

    return out
